# baseline (speedup 1.0000x reference)
import functools

import jax
import jax.numpy as jnp
from jax import lax
from jax.experimental import pallas as pl
from jax.experimental.pallas import tpu as pltpu
from jax.experimental.pallas import tpu_sc as plsc

_N = 10000
_E = 320000
_G = 64
_NTILES = 16
_RPT = _N // _NTILES


def _edge_encoder(edge_attr, W_e1, b_e1, W_e2, b_e2, W_c1e, b_c1e):
    BE = 2560

    def body(att, we1, be1, we2, be2, wc1e, bc1e, ea_out, l1_out):
        bf = jnp.bfloat16
        t = jnp.maximum(
            jnp.dot(att[...].astype(bf), we1[...].astype(bf),
                    preferred_element_type=jnp.float32) + be1[...], 0.0)
        ea = jnp.dot(t.astype(bf), we2[...].astype(bf),
                     preferred_element_type=jnp.float32) + be2[...]
        l1 = jnp.dot(ea.astype(bf), wc1e[...].astype(bf),
                     preferred_element_type=jnp.float32) + bc1e[...]
        ea_out[0] = ea[:, :128]
        ea_out[1] = ea[:, 128:]
        l1_out[...] = l1

    return pl.pallas_call(
        body,
        grid=(_E // BE,),
        in_specs=[
            pl.BlockSpec((BE, 16), lambda i: (i, 0)),
            pl.BlockSpec((16, 256), lambda i: (0, 0)),
            pl.BlockSpec((1, 256), lambda i: (0, 0)),
            pl.BlockSpec((256, 256), lambda i: (0, 0)),
            pl.BlockSpec((1, 256), lambda i: (0, 0)),
            pl.BlockSpec((256, 128), lambda i: (0, 0)),
            pl.BlockSpec((1, 128), lambda i: (0, 0)),
        ],
        out_specs=[
            pl.BlockSpec((2, BE, 128), lambda i: (0, i, 0)),
            pl.BlockSpec((BE, 128), lambda i: (i, 0)),
        ],
        out_shape=[
            jax.ShapeDtypeStruct((2, _E, 128), jnp.float32),
            jax.ShapeDtypeStruct((_E, 128), jnp.float32),
        ],
    )(edge_attr, W_e1, b_e1.reshape(1, -1), W_e2, b_e2.reshape(1, -1),
      W_c1e, b_c1e.reshape(1, -1))


def _scatter_phase(table, msg, src2, dst2, feat_split):
    if feat_split:
        n_groups = _E // 4096
        core_sub0 = 0
    else:
        n_groups = _E // 8192
        core_sub0 = n_groups * 32
    nfull = n_groups // _NTILES
    nextra = n_groups - nfull * _NTILES
    rem_sub = (_E // 128) - 4
    NRND = 2
    RN = _N // NRND
    AR = RN + 8
    TRASH = RN
    zpt = 312
    PR = 128
    NBUF = 5
    mesh = plsc.VectorSubcoreMesh(core_axis_name="c", subcore_axis_name="s")

    @functools.partial(
        pl.kernel,
        out_type=jax.ShapeDtypeStruct((2, _N, 128), jnp.float32),
        mesh=mesh,
        scratch_types=[
            pltpu.VMEM((32, 128), jnp.int32),
            pltpu.VMEM((32, 128), jnp.int32),
            [pltpu.VMEM((PR, 128), jnp.float32)] * NBUF,
            pltpu.VMEM_SHARED((AR, 128), jnp.float32),
            [pltpu.SemaphoreType.DMA] * NBUF,
            [pltpu.SemaphoreType.DMA] * NBUF,
            [pltpu.SemaphoreType.DMA] * NBUF,
        ],
    )
    def mp(table_h, msg_h, src_h, dst_h, out_h, src_v, dst_v,
           bufs, acc, msems, gsems, ssems):
        c = lax.axis_index("c")
        s = lax.axis_index("s")
        shift = c * _N
        row0 = zpt * s

        def load_idx(base_sub, ksubs, nb):
            pltpu.sync_copy(src_h.at[pl.ds(base_sub, ksubs)], src_v.at[pl.ds(0, ksubs)])
            pltpu.sync_copy(dst_h.at[pl.ds(base_sub, ksubs)], dst_v.at[pl.ds(0, ksubs)])

            def sh(j, carry):
                for k2 in range(8):
                    if feat_split:
                        src_v[j, pl.ds(16 * k2, 16)] = src_v[j, pl.ds(16 * k2, 16)] + shift
                    dv = dst_v[j, pl.ds(16 * k2, 16)]
                    ok = jnp.logical_and(dv >= nb, dv < nb + RN)
                    dst_v[j, pl.ds(16 * k2, 16)] = jnp.where(ok, dv - nb, TRASH)
                return carry

            lax.fori_loop(0, ksubs, sh, None)

        def fire_msg(base_sub, p):
            b = bufs[p % NBUF]
            base_e = base_sub * 128 + PR * p
            if feat_split:
                return pltpu.async_copy(msg_h.at[c, pl.ds(base_e, PR)], b,
                                        msems[p % NBUF])
            return pltpu.async_copy(msg_h.at[pl.ds(base_e, PR)], b, msems[p % NBUF])

        def fire_gathers(p):
            return [
                pltpu.async_copy(table_h.at[src_v.at[p]],
                                 bufs[p % NBUF], gsems[p % NBUF], add=True)
            ]

        def fire_scatters(p):
            return [
                pltpu.async_copy(bufs[p % NBUF], acc.at[dst_v.at[p]],
                                 ssems[p % NBUF], add=True)
            ]

        def relu_pass(p):
            b = bufs[p % NBUF]

            def rl(r, carry):
                for q in range(4):
                    for k2 in range(8):
                        b[r * 4 + q, pl.ds(16 * k2, 16)] = jnp.maximum(
                            b[r * 4 + q, pl.ds(16 * k2, 16)], 0.0)
                return carry

            lax.fori_loop(0, PR // 4, rl, None)

        def process_group(base_sub, nsubs, nb):
            load_idx(base_sub, nsubs, nb)
            npass = nsubs
            m = [None] * npass
            g = [None] * npass
            sc = [None] * npass
            for p in range(min(NBUF, npass)):
                m[p] = fire_msg(base_sub, p)
            m[0].wait()
            g[0] = fire_gathers(0)
            for p in range(npass):
                nxt = p + 1
                if nxt < npass:
                    if nxt >= NBUF:
                        for dd in sc[nxt - NBUF]:
                            dd.wait()
                        m[nxt] = fire_msg(base_sub, nxt)
                    m[nxt].wait()
                    g[nxt] = fire_gathers(nxt)
                for dd in g[p]:
                    dd.wait()
                relu_pass(p)
                sc[p] = fire_scatters(p)
            for p in range(max(0, npass - NBUF), npass):
                for dd in sc[p]:
                    dd.wait()

        for rnd in range(NRND):
            nb = RN * rnd

            def zrow(r, carry):
                for q in range(4):
                    for k2 in range(8):
                        bufs[0][r * 4 + q, pl.ds(16 * k2, 16)] = jnp.zeros(
                            (16,), jnp.float32)
                return carry

            lax.fori_loop(0, PR // 4, zrow, None)
            pltpu.sync_copy(bufs[0], acc.at[pl.ds(row0, PR)])
            pltpu.sync_copy(bufs[0], acc.at[pl.ds(row0 + PR, PR)])
            pltpu.sync_copy(bufs[0].at[pl.ds(0, zpt - 2 * PR)],
                            acc.at[pl.ds(row0 + 2 * PR, zpt - 2 * PR)])

            @pl.when(s == 0)
            def _():
                pltpu.sync_copy(bufs[0].at[pl.ds(0, 16)], acc.at[pl.ds(AR - 16, 16)])

            plsc.subcore_barrier()

            def chunk(k, carry):
                g = s + _NTILES * k
                base_sub = (0 if feat_split else core_sub0 * c) + 32 * g
                process_group(base_sub, 32, nb)
                return carry

            nk = jnp.where(s < nextra, nfull + 1, nfull)
            lax.fori_loop(0, nk, chunk, None)

            rem_here = (s == 0) if feat_split else jnp.logical_and(s == 0, c == 0)

            @pl.when(rem_here)
            def _():
                process_group(rem_sub, 4, nb)

            plsc.subcore_barrier()
            pltpu.sync_copy(acc.at[pl.ds(row0, zpt)],
                            out_h.at[c, pl.ds(nb + row0, zpt)])

            @pl.when(s == 0)
            def _():
                pltpu.sync_copy(acc.at[pl.ds(16 * zpt, 8)],
                                out_h.at[c, pl.ds(nb + 16 * zpt, 8)])

            plsc.subcore_barrier()

    return mp(table, msg, src2, dst2)


def _node_mlp1(x, acc1, W11, b11, W12, b12):
    BN = 1000

    def body(x_ref, a_ref, w11, b11r, w12, b12r, h_out):
        hin = x_ref[...] + a_ref[0] + a_ref[1]
        t = jnp.maximum(
            jnp.dot(hin, w11[...], preferred_element_type=jnp.float32) + b11r[...], 0.0)
        h = jnp.maximum(
            jnp.dot(t, w12[...], preferred_element_type=jnp.float32) + b12r[...], 0.0)
        h_out[0] = h[:, :128]
        h_out[1] = h[:, 128:]

    return pl.pallas_call(
        body,
        grid=(_N // BN,),
        in_specs=[
            pl.BlockSpec((BN, 128), lambda i: (i, 0)),
            pl.BlockSpec((2, BN, 128), lambda i: (0, i, 0)),
            pl.BlockSpec((128, 256), lambda i: (0, 0)),
            pl.BlockSpec((1, 256), lambda i: (0, 0)),
            pl.BlockSpec((256, 256), lambda i: (0, 0)),
            pl.BlockSpec((1, 256), lambda i: (0, 0)),
        ],
        out_specs=pl.BlockSpec((2, BN, 128), lambda i: (0, i, 0)),
        out_shape=jax.ShapeDtypeStruct((2, _N, 128), jnp.float32),
    )(x, acc1, W11, b11.reshape(1, -1), W12, b12.reshape(1, -1))


def _final_stage(h2way, acc2, batch, W21, b21, W22, b22, W_out, b_out):
    BN = 1000
    nblk = _N // BN

    def body(h_ref, a_ref, b_ref, w21, b21r, w22, b22r, wout, boutr,
             out_ref, sums, cnts):
        i = pl.program_id(0)

        @pl.when(i == 0)
        def _():
            sums[...] = jnp.zeros_like(sums)
            cnts[...] = jnp.zeros_like(cnts)

        hin = (jnp.concatenate([h_ref[0], h_ref[1]], axis=-1)
               + jnp.concatenate([a_ref[0], a_ref[1]], axis=-1))
        t = jnp.maximum(
            jnp.dot(hin, w21[...], preferred_element_type=jnp.float32) + b21r[...], 0.0)
        h2 = jnp.maximum(
            jnp.dot(t, w22[...], preferred_element_type=jnp.float32) + b22r[...], 0.0)
        bidx = b_ref[0, 0]
        onehot = (bidx[:, None] == lax.broadcasted_iota(jnp.int32, (BN, _G), 1)
                  ).astype(jnp.float32)
        sums[...] = sums[...] + lax.dot_general(
            onehot, h2, (((0,), (0,)), ((), ())), preferred_element_type=jnp.float32)
        cnts[...] = cnts[...] + jnp.sum(onehot, axis=0)[:, None]

        @pl.when(i == nblk - 1)
        def _():
            pooled = sums[...] / jnp.maximum(cnts[:, 0:1], 1.0)
            out_ref[...] = jnp.dot(
                pooled, wout[...], preferred_element_type=jnp.float32) + boutr[...]

    return pl.pallas_call(
        body,
        grid=(nblk,),
        in_specs=[
            pl.BlockSpec((2, BN, 128), lambda i: (0, i, 0)),
            pl.BlockSpec((2, BN, 128), lambda i: (0, i, 0)),
            pl.BlockSpec((1, 1, BN), lambda i: (i, 0, 0)),
            pl.BlockSpec((256, 256), lambda i: (0, 0)),
            pl.BlockSpec((1, 256), lambda i: (0, 0)),
            pl.BlockSpec((256, 256), lambda i: (0, 0)),
            pl.BlockSpec((1, 256), lambda i: (0, 0)),
            pl.BlockSpec((256, 10), lambda i: (0, 0)),
            pl.BlockSpec((1, 10), lambda i: (0, 0)),
        ],
        out_specs=pl.BlockSpec((_G, 10), lambda i: (0, 0)),
        out_shape=jax.ShapeDtypeStruct((_G, 10), jnp.float32),
        scratch_shapes=[
            pltpu.VMEM((_G, 256), jnp.float32),
            pltpu.VMEM((_G, 128), jnp.float32),
        ],
    )(h2way, acc2, batch.reshape(nblk, 1, BN), W21, b21.reshape(1, -1),
      W22, b22.reshape(1, -1), W_out, b_out.reshape(1, -1))


def kernel(x, edge_index, edge_attr, batch, W_e1, b_e1, W_e2, b_e2, W_c1e, b_c1e,
           W11, b11, W12, b12, W21, b21, W22, b22, W_out, b_out):
    src2 = edge_index[0].reshape(_E // 128, 128)
    dst2 = edge_index[1].reshape(_E // 128, 128)

    ea2, lin1 = _edge_encoder(edge_attr, W_e1, b_e1, W_e2, b_e2, W_c1e, b_c1e)
    acc1 = _scatter_phase(x, lin1, src2, dst2, feat_split=False)
    h2way = _node_mlp1(x, acc1, W11, b11, W12, b12)
    hflat = h2way.reshape(2 * _N, 128)
    acc2 = _scatter_phase(hflat, ea2, src2, dst2, feat_split=True)
    return _final_stage(h2way, acc2, batch, W21, b21, W22, b22, W_out, b_out)


# speedup vs baseline: 1.0314x; 1.0314x over previous
import functools

import jax
import jax.numpy as jnp
from jax import lax
from jax.experimental import pallas as pl
from jax.experimental.pallas import tpu as pltpu
from jax.experimental.pallas import tpu_sc as plsc

_N = 10000
_E = 320000
_G = 64
_NTILES = 16
_RPT = _N // _NTILES


def _edge_encoder(edge_attr, W_e1, b_e1, W_e2, b_e2, W_c1e, b_c1e):
    BE = 2560

    def body(att, we1, be1, we2, be2, wc1e, bc1e, ea_out, l1_out):
        bf = jnp.bfloat16
        t = jnp.maximum(
            jnp.dot(att[...].astype(bf), we1[...].astype(bf),
                    preferred_element_type=jnp.float32) + be1[...], 0.0)
        ea = jnp.dot(t.astype(bf), we2[...].astype(bf),
                     preferred_element_type=jnp.float32) + be2[...]
        l1 = jnp.dot(ea.astype(bf), wc1e[...].astype(bf),
                     preferred_element_type=jnp.float32) + bc1e[...]
        ea_out[0] = ea[:, :128]
        ea_out[1] = ea[:, 128:]
        l1_out[...] = l1

    return pl.pallas_call(
        body,
        grid=(_E // BE,),
        in_specs=[
            pl.BlockSpec((BE, 16), lambda i: (i, 0)),
            pl.BlockSpec((16, 256), lambda i: (0, 0)),
            pl.BlockSpec((1, 256), lambda i: (0, 0)),
            pl.BlockSpec((256, 256), lambda i: (0, 0)),
            pl.BlockSpec((1, 256), lambda i: (0, 0)),
            pl.BlockSpec((256, 128), lambda i: (0, 0)),
            pl.BlockSpec((1, 128), lambda i: (0, 0)),
        ],
        out_specs=[
            pl.BlockSpec((2, BE, 128), lambda i: (0, i, 0)),
            pl.BlockSpec((BE, 128), lambda i: (i, 0)),
        ],
        out_shape=[
            jax.ShapeDtypeStruct((2, _E, 128), jnp.float32),
            jax.ShapeDtypeStruct((_E, 128), jnp.float32),
        ],
    )(edge_attr, W_e1, b_e1.reshape(1, -1), W_e2, b_e2.reshape(1, -1),
      W_c1e, b_c1e.reshape(1, -1))


def _scatter_phase(table, msg, src2, dst2, feat_split):
    if feat_split:
        n_groups = _E // 2048
        core_sub0 = 0
    else:
        n_groups = _E // 4096
        core_sub0 = n_groups * 16
    nfull = n_groups // _NTILES
    nextra = n_groups - nfull * _NTILES
    rem_sub = (_E // 128) - 4
    NRND = 2
    RN = _N // NRND
    AR = RN + 8
    TRASH = RN
    zpt = 312
    PR = 128
    NBUF = 5
    mesh = plsc.VectorSubcoreMesh(core_axis_name="c", subcore_axis_name="s")

    @functools.partial(
        pl.kernel,
        out_type=jax.ShapeDtypeStruct((2, _N, 128), jnp.float32),
        mesh=mesh,
        scratch_types=[
            pltpu.VMEM((32, 128), jnp.int32),
            pltpu.VMEM((32, 128), jnp.int32),
            [pltpu.VMEM((PR, 128), jnp.float32)] * NBUF,
            pltpu.VMEM_SHARED((AR, 128), jnp.float32),
            [pltpu.SemaphoreType.DMA] * NBUF,
            [pltpu.SemaphoreType.DMA] * NBUF,
            [pltpu.SemaphoreType.DMA] * NBUF,
        ],
    )
    def mp(table_h, msg_h, src_h, dst_h, out_h, src_v, dst_v,
           bufs, acc, msems, gsems, ssems):
        c = lax.axis_index("c")
        s = lax.axis_index("s")
        shift = c * _N
        row0 = zpt * s

        def load_idx(base_sub, ksubs, nb):
            pltpu.sync_copy(src_h.at[pl.ds(base_sub, ksubs)], src_v.at[pl.ds(0, ksubs)])
            pltpu.sync_copy(dst_h.at[pl.ds(base_sub, ksubs)], dst_v.at[pl.ds(0, ksubs)])

            def sh(j, carry):
                for k2 in range(8):
                    if feat_split:
                        src_v[j, pl.ds(16 * k2, 16)] = src_v[j, pl.ds(16 * k2, 16)] + shift
                    dv = dst_v[j, pl.ds(16 * k2, 16)]
                    ok = jnp.logical_and(dv >= nb, dv < nb + RN)
                    dst_v[j, pl.ds(16 * k2, 16)] = jnp.where(ok, dv - nb, TRASH)
                return carry

            lax.fori_loop(0, ksubs, sh, None)

        def fire_msg(base_sub, p):
            b = bufs[p % NBUF]
            base_e = base_sub * 128 + PR * p
            if feat_split:
                return pltpu.async_copy(msg_h.at[c, pl.ds(base_e, PR)], b,
                                        msems[p % NBUF])
            return pltpu.async_copy(msg_h.at[pl.ds(base_e, PR)], b, msems[p % NBUF])

        def fire_gathers(p):
            return [
                pltpu.async_copy(table_h.at[src_v.at[p]],
                                 bufs[p % NBUF], gsems[p % NBUF], add=True)
            ]

        def fire_scatters(p):
            return [
                pltpu.async_copy(bufs[p % NBUF], acc.at[dst_v.at[p]],
                                 ssems[p % NBUF], add=True)
            ]

        def relu_pass(p):
            b = bufs[p % NBUF]

            def rl(r, carry):
                for q in range(4):
                    for k2 in range(8):
                        b[r * 4 + q, pl.ds(16 * k2, 16)] = jnp.maximum(
                            b[r * 4 + q, pl.ds(16 * k2, 16)], 0.0)
                return carry

            lax.fori_loop(0, PR // 4, rl, None)

        def process_group(base_sub, nsubs, nb):
            load_idx(base_sub, nsubs, nb)
            npass = nsubs
            m = [None] * npass
            g = [None] * npass
            sc = [None] * npass
            for p in range(min(NBUF, npass)):
                m[p] = fire_msg(base_sub, p)
            m[0].wait()
            g[0] = fire_gathers(0)
            for p in range(npass):
                nxt = p + 1
                if nxt < npass:
                    if nxt >= NBUF:
                        for dd in sc[nxt - NBUF]:
                            dd.wait()
                        m[nxt] = fire_msg(base_sub, nxt)
                    m[nxt].wait()
                    g[nxt] = fire_gathers(nxt)
                for dd in g[p]:
                    dd.wait()
                relu_pass(p)
                sc[p] = fire_scatters(p)
            for p in range(max(0, npass - NBUF), npass):
                for dd in sc[p]:
                    dd.wait()

        for rnd in range(NRND):
            nb = RN * rnd

            def zrow(r, carry):
                for q in range(4):
                    for k2 in range(8):
                        bufs[0][r * 4 + q, pl.ds(16 * k2, 16)] = jnp.zeros(
                            (16,), jnp.float32)
                return carry

            lax.fori_loop(0, PR // 4, zrow, None)
            pltpu.sync_copy(bufs[0], acc.at[pl.ds(row0, PR)])
            pltpu.sync_copy(bufs[0], acc.at[pl.ds(row0 + PR, PR)])
            pltpu.sync_copy(bufs[0].at[pl.ds(0, zpt - 2 * PR)],
                            acc.at[pl.ds(row0 + 2 * PR, zpt - 2 * PR)])

            @pl.when(s == 0)
            def _():
                pltpu.sync_copy(bufs[0].at[pl.ds(0, 16)], acc.at[pl.ds(AR - 16, 16)])

            plsc.subcore_barrier()

            def chunk(k, carry):
                g = s + _NTILES * k
                base_sub = (0 if feat_split else core_sub0 * c) + 16 * g
                process_group(base_sub, 16, nb)
                return carry

            nk = jnp.where(s < nextra, nfull + 1, nfull)
            lax.fori_loop(0, nk, chunk, None)

            rem_here = (s == 0) if feat_split else jnp.logical_and(s == 0, c == 0)

            @pl.when(rem_here)
            def _():
                process_group(rem_sub, 4, nb)

            plsc.subcore_barrier()
            pltpu.sync_copy(acc.at[pl.ds(row0, zpt)],
                            out_h.at[c, pl.ds(nb + row0, zpt)])

            @pl.when(s == 0)
            def _():
                pltpu.sync_copy(acc.at[pl.ds(16 * zpt, 8)],
                                out_h.at[c, pl.ds(nb + 16 * zpt, 8)])

            plsc.subcore_barrier()

    return mp(table, msg, src2, dst2)


def _node_mlp1(x, acc1, W11, b11, W12, b12):
    BN = 1000

    def body(x_ref, a_ref, w11, b11r, w12, b12r, h_out):
        hin = x_ref[...] + a_ref[0] + a_ref[1]
        t = jnp.maximum(
            jnp.dot(hin, w11[...], preferred_element_type=jnp.float32) + b11r[...], 0.0)
        h = jnp.maximum(
            jnp.dot(t, w12[...], preferred_element_type=jnp.float32) + b12r[...], 0.0)
        h_out[0] = h[:, :128]
        h_out[1] = h[:, 128:]

    return pl.pallas_call(
        body,
        grid=(_N // BN,),
        in_specs=[
            pl.BlockSpec((BN, 128), lambda i: (i, 0)),
            pl.BlockSpec((2, BN, 128), lambda i: (0, i, 0)),
            pl.BlockSpec((128, 256), lambda i: (0, 0)),
            pl.BlockSpec((1, 256), lambda i: (0, 0)),
            pl.BlockSpec((256, 256), lambda i: (0, 0)),
            pl.BlockSpec((1, 256), lambda i: (0, 0)),
        ],
        out_specs=pl.BlockSpec((2, BN, 128), lambda i: (0, i, 0)),
        out_shape=jax.ShapeDtypeStruct((2, _N, 128), jnp.float32),
    )(x, acc1, W11, b11.reshape(1, -1), W12, b12.reshape(1, -1))


def _final_stage(h2way, acc2, batch, W21, b21, W22, b22, W_out, b_out):
    BN = 1000
    nblk = _N // BN

    def body(h_ref, a_ref, b_ref, w21, b21r, w22, b22r, wout, boutr,
             out_ref, sums, cnts):
        i = pl.program_id(0)

        @pl.when(i == 0)
        def _():
            sums[...] = jnp.zeros_like(sums)
            cnts[...] = jnp.zeros_like(cnts)

        hin = (jnp.concatenate([h_ref[0], h_ref[1]], axis=-1)
               + jnp.concatenate([a_ref[0], a_ref[1]], axis=-1))
        t = jnp.maximum(
            jnp.dot(hin, w21[...], preferred_element_type=jnp.float32) + b21r[...], 0.0)
        h2 = jnp.maximum(
            jnp.dot(t, w22[...], preferred_element_type=jnp.float32) + b22r[...], 0.0)
        bidx = b_ref[0, 0]
        onehot = (bidx[:, None] == lax.broadcasted_iota(jnp.int32, (BN, _G), 1)
                  ).astype(jnp.float32)
        sums[...] = sums[...] + lax.dot_general(
            onehot, h2, (((0,), (0,)), ((), ())), preferred_element_type=jnp.float32)
        cnts[...] = cnts[...] + jnp.sum(onehot, axis=0)[:, None]

        @pl.when(i == nblk - 1)
        def _():
            pooled = sums[...] / jnp.maximum(cnts[:, 0:1], 1.0)
            out_ref[...] = jnp.dot(
                pooled, wout[...], preferred_element_type=jnp.float32) + boutr[...]

    return pl.pallas_call(
        body,
        grid=(nblk,),
        in_specs=[
            pl.BlockSpec((2, BN, 128), lambda i: (0, i, 0)),
            pl.BlockSpec((2, BN, 128), lambda i: (0, i, 0)),
            pl.BlockSpec((1, 1, BN), lambda i: (i, 0, 0)),
            pl.BlockSpec((256, 256), lambda i: (0, 0)),
            pl.BlockSpec((1, 256), lambda i: (0, 0)),
            pl.BlockSpec((256, 256), lambda i: (0, 0)),
            pl.BlockSpec((1, 256), lambda i: (0, 0)),
            pl.BlockSpec((256, 10), lambda i: (0, 0)),
            pl.BlockSpec((1, 10), lambda i: (0, 0)),
        ],
        out_specs=pl.BlockSpec((_G, 10), lambda i: (0, 0)),
        out_shape=jax.ShapeDtypeStruct((_G, 10), jnp.float32),
        scratch_shapes=[
            pltpu.VMEM((_G, 256), jnp.float32),
            pltpu.VMEM((_G, 128), jnp.float32),
        ],
    )(h2way, acc2, batch.reshape(nblk, 1, BN), W21, b21.reshape(1, -1),
      W22, b22.reshape(1, -1), W_out, b_out.reshape(1, -1))


def kernel(x, edge_index, edge_attr, batch, W_e1, b_e1, W_e2, b_e2, W_c1e, b_c1e,
           W11, b11, W12, b12, W21, b21, W22, b22, W_out, b_out):
    src2 = edge_index[0].reshape(_E // 128, 128)
    dst2 = edge_index[1].reshape(_E // 128, 128)

    ea2, lin1 = _edge_encoder(edge_attr, W_e1, b_e1, W_e2, b_e2, W_c1e, b_c1e)
    acc1 = _scatter_phase(x, lin1, src2, dst2, feat_split=False)
    h2way = _node_mlp1(x, acc1, W11, b11, W12, b12)
    hflat = h2way.reshape(2 * _N, 128)
    acc2 = _scatter_phase(hflat, ea2, src2, dst2, feat_split=True)
    return _final_stage(h2way, acc2, batch, W21, b21, W22, b22, W_out, b_out)


# speedup vs baseline: 1.0427x; 1.0110x over previous
import functools

import jax
import jax.numpy as jnp
from jax import lax
from jax.experimental import pallas as pl
from jax.experimental.pallas import tpu as pltpu
from jax.experimental.pallas import tpu_sc as plsc

_N = 10000
_E = 320000
_G = 64
_NTILES = 16
_RPT = _N // _NTILES


def _edge_encoder(edge_attr, W_e1, b_e1, W_e2, b_e2, W_c1e, b_c1e):
    BE = 2560

    def body(att, we1, be1, we2, be2, wc1e, bc1e, ea_out, l1_out):
        bf = jnp.bfloat16
        t = jnp.maximum(
            jnp.dot(att[...].astype(bf), we1[...].astype(bf),
                    preferred_element_type=jnp.float32) + be1[...], 0.0)
        ea = jnp.dot(t.astype(bf), we2[...].astype(bf),
                     preferred_element_type=jnp.float32) + be2[...]
        l1 = jnp.dot(ea.astype(bf), wc1e[...].astype(bf),
                     preferred_element_type=jnp.float32) + bc1e[...]
        ea_out[0] = ea[:, :128]
        ea_out[1] = ea[:, 128:]
        l1_out[...] = l1

    return pl.pallas_call(
        body,
        grid=(_E // BE,),
        in_specs=[
            pl.BlockSpec((BE, 16), lambda i: (i, 0)),
            pl.BlockSpec((16, 256), lambda i: (0, 0)),
            pl.BlockSpec((1, 256), lambda i: (0, 0)),
            pl.BlockSpec((256, 256), lambda i: (0, 0)),
            pl.BlockSpec((1, 256), lambda i: (0, 0)),
            pl.BlockSpec((256, 128), lambda i: (0, 0)),
            pl.BlockSpec((1, 128), lambda i: (0, 0)),
        ],
        out_specs=[
            pl.BlockSpec((2, BE, 128), lambda i: (0, i, 0)),
            pl.BlockSpec((BE, 128), lambda i: (i, 0)),
        ],
        out_shape=[
            jax.ShapeDtypeStruct((2, _E, 128), jnp.float32),
            jax.ShapeDtypeStruct((_E, 128), jnp.float32),
        ],
    )(edge_attr, W_e1, b_e1.reshape(1, -1), W_e2, b_e2.reshape(1, -1),
      W_c1e, b_c1e.reshape(1, -1))


def _scatter_phase(table, msg, src2, dst2, feat_split):
    if feat_split:
        n_groups = _E // 2048
        core_sub0 = 0
    else:
        n_groups = _E // 4096
        core_sub0 = n_groups * 16
    nfull = n_groups // _NTILES
    nextra = n_groups - nfull * _NTILES
    rem_sub = (_E // 128) - 4
    NRND = 2
    RN = _N // NRND
    AR = RN + 8
    TRASH = RN
    zpt = 312
    PR = 128
    NBUF = 5
    mesh = plsc.VectorSubcoreMesh(core_axis_name="c", subcore_axis_name="s")

    @functools.partial(
        pl.kernel,
        out_type=jax.ShapeDtypeStruct((2, _N, 128), jnp.float32),
        mesh=mesh,
        scratch_types=[
            pltpu.VMEM((32, 128), jnp.int32),
            pltpu.VMEM((32, 128), jnp.int32),
            [pltpu.VMEM((PR, 128), jnp.float32)] * NBUF,
            pltpu.VMEM_SHARED((AR, 128), jnp.float32),
            [pltpu.SemaphoreType.DMA] * NBUF,
            [pltpu.SemaphoreType.DMA] * NBUF,
            [pltpu.SemaphoreType.DMA] * NBUF,
        ],
    )
    def mp(table_h, msg_h, src_h, dst_h, out_h, src_v, dst_v,
           bufs, acc, msems, gsems, ssems):
        c = lax.axis_index("c")
        s = lax.axis_index("s")
        shift = c * _N
        row0 = zpt * s

        def load_idx(base_sub, ksubs, nb):
            pltpu.sync_copy(src_h.at[pl.ds(base_sub, ksubs)], src_v.at[pl.ds(0, ksubs)])
            pltpu.sync_copy(dst_h.at[pl.ds(base_sub, ksubs)], dst_v.at[pl.ds(0, ksubs)])

            def sh(j, carry):
                for k2 in range(8):
                    if feat_split:
                        src_v[j, pl.ds(16 * k2, 16)] = src_v[j, pl.ds(16 * k2, 16)] + shift
                    dv = dst_v[j, pl.ds(16 * k2, 16)]
                    ok = jnp.logical_and(dv >= nb, dv < nb + RN)
                    dst_v[j, pl.ds(16 * k2, 16)] = jnp.where(ok, dv - nb, TRASH)
                return carry

            lax.fori_loop(0, ksubs, sh, None)

        def fire_msg(base_sub, p):
            b = bufs[p % NBUF]
            base_e = base_sub * 128 + PR * p
            if feat_split:
                return pltpu.async_copy(msg_h.at[c, pl.ds(base_e, PR)], b,
                                        msems[p % NBUF])
            return pltpu.async_copy(msg_h.at[pl.ds(base_e, PR)], b, msems[p % NBUF])

        def fire_gathers(p):
            return [
                pltpu.async_copy(table_h.at[src_v.at[p]],
                                 bufs[p % NBUF], gsems[p % NBUF], add=True)
            ]

        def fire_scatters(p):
            return [
                pltpu.async_copy(bufs[p % NBUF], acc.at[dst_v.at[p]],
                                 ssems[p % NBUF], add=True)
            ]

        def relu_pass(p):
            b = bufs[p % NBUF]

            def rl(r, carry):
                for q in range(4):
                    for k2 in range(8):
                        b[r * 4 + q, pl.ds(16 * k2, 16)] = jnp.maximum(
                            b[r * 4 + q, pl.ds(16 * k2, 16)], 0.0)
                return carry

            lax.fori_loop(0, PR // 4, rl, None)

        def process_group(base_sub, nsubs, nb):
            load_idx(base_sub, nsubs, nb)
            npass = nsubs
            m = [None] * npass
            g = [None] * npass
            sc = [None] * npass
            for p in range(min(NBUF, npass)):
                m[p] = fire_msg(base_sub, p)
            m[0].wait()
            g[0] = fire_gathers(0)
            for p in range(npass):
                nxt = p + 1
                if nxt < npass:
                    if nxt >= NBUF:
                        for dd in sc[nxt - NBUF]:
                            dd.wait()
                        m[nxt] = fire_msg(base_sub, nxt)
                    m[nxt].wait()
                    g[nxt] = fire_gathers(nxt)
                for dd in g[p]:
                    dd.wait()
                relu_pass(p)
                sc[p] = fire_scatters(p)
            for p in range(max(0, npass - NBUF), npass):
                for dd in sc[p]:
                    dd.wait()

        for rnd in range(NRND):
            nb = RN * rnd

            def zrow(r, carry):
                for q in range(4):
                    for k2 in range(8):
                        bufs[0][r * 4 + q, pl.ds(16 * k2, 16)] = jnp.zeros(
                            (16,), jnp.float32)
                return carry

            lax.fori_loop(0, PR // 4, zrow, None)
            pltpu.sync_copy(bufs[0], acc.at[pl.ds(row0, PR)])
            pltpu.sync_copy(bufs[0], acc.at[pl.ds(row0 + PR, PR)])
            pltpu.sync_copy(bufs[0].at[pl.ds(0, zpt - 2 * PR)],
                            acc.at[pl.ds(row0 + 2 * PR, zpt - 2 * PR)])

            @pl.when(s == 0)
            def _():
                pltpu.sync_copy(bufs[0].at[pl.ds(0, 16)], acc.at[pl.ds(AR - 16, 16)])

            plsc.subcore_barrier()

            def chunk(k, carry):
                g = s + _NTILES * k
                base_sub = (0 if feat_split else core_sub0 * c) + 16 * g
                process_group(base_sub, 16, nb)
                return carry

            nk = jnp.where(s < nextra, nfull + 1, nfull)
            lax.fori_loop(0, nk, chunk, None)

            rem_here = (s == 15) if feat_split else jnp.logical_and(s == 15, c == 0)

            @pl.when(rem_here)
            def _():
                process_group(rem_sub, 4, nb)

            plsc.subcore_barrier()
            pltpu.sync_copy(acc.at[pl.ds(row0, zpt)],
                            out_h.at[c, pl.ds(nb + row0, zpt)])

            @pl.when(s == 0)
            def _():
                pltpu.sync_copy(acc.at[pl.ds(16 * zpt, 8)],
                                out_h.at[c, pl.ds(nb + 16 * zpt, 8)])

            plsc.subcore_barrier()

    return mp(table, msg, src2, dst2)


def _node_mlp1(x, acc1, W11, b11, W12, b12):
    BN = 1000

    def body(x_ref, a_ref, w11, b11r, w12, b12r, h_out):
        hin = x_ref[...] + a_ref[0] + a_ref[1]
        t = jnp.maximum(
            jnp.dot(hin, w11[...], preferred_element_type=jnp.float32) + b11r[...], 0.0)
        h = jnp.maximum(
            jnp.dot(t, w12[...], preferred_element_type=jnp.float32) + b12r[...], 0.0)
        h_out[0] = h[:, :128]
        h_out[1] = h[:, 128:]

    return pl.pallas_call(
        body,
        grid=(_N // BN,),
        in_specs=[
            pl.BlockSpec((BN, 128), lambda i: (i, 0)),
            pl.BlockSpec((2, BN, 128), lambda i: (0, i, 0)),
            pl.BlockSpec((128, 256), lambda i: (0, 0)),
            pl.BlockSpec((1, 256), lambda i: (0, 0)),
            pl.BlockSpec((256, 256), lambda i: (0, 0)),
            pl.BlockSpec((1, 256), lambda i: (0, 0)),
        ],
        out_specs=pl.BlockSpec((2, BN, 128), lambda i: (0, i, 0)),
        out_shape=jax.ShapeDtypeStruct((2, _N, 128), jnp.float32),
    )(x, acc1, W11, b11.reshape(1, -1), W12, b12.reshape(1, -1))


def _final_stage(h2way, acc2, batch, W21, b21, W22, b22, W_out, b_out):
    BN = 1000
    nblk = _N // BN

    def body(h_ref, a_ref, b_ref, w21, b21r, w22, b22r, wout, boutr,
             out_ref, sums, cnts):
        i = pl.program_id(0)

        @pl.when(i == 0)
        def _():
            sums[...] = jnp.zeros_like(sums)
            cnts[...] = jnp.zeros_like(cnts)

        hin = (jnp.concatenate([h_ref[0], h_ref[1]], axis=-1)
               + jnp.concatenate([a_ref[0], a_ref[1]], axis=-1))
        t = jnp.maximum(
            jnp.dot(hin, w21[...], preferred_element_type=jnp.float32) + b21r[...], 0.0)
        h2 = jnp.maximum(
            jnp.dot(t, w22[...], preferred_element_type=jnp.float32) + b22r[...], 0.0)
        bidx = b_ref[0, 0]
        onehot = (bidx[:, None] == lax.broadcasted_iota(jnp.int32, (BN, _G), 1)
                  ).astype(jnp.float32)
        sums[...] = sums[...] + lax.dot_general(
            onehot, h2, (((0,), (0,)), ((), ())), preferred_element_type=jnp.float32)
        cnts[...] = cnts[...] + jnp.sum(onehot, axis=0)[:, None]

        @pl.when(i == nblk - 1)
        def _():
            pooled = sums[...] / jnp.maximum(cnts[:, 0:1], 1.0)
            out_ref[...] = jnp.dot(
                pooled, wout[...], preferred_element_type=jnp.float32) + boutr[...]

    return pl.pallas_call(
        body,
        grid=(nblk,),
        in_specs=[
            pl.BlockSpec((2, BN, 128), lambda i: (0, i, 0)),
            pl.BlockSpec((2, BN, 128), lambda i: (0, i, 0)),
            pl.BlockSpec((1, 1, BN), lambda i: (i, 0, 0)),
            pl.BlockSpec((256, 256), lambda i: (0, 0)),
            pl.BlockSpec((1, 256), lambda i: (0, 0)),
            pl.BlockSpec((256, 256), lambda i: (0, 0)),
            pl.BlockSpec((1, 256), lambda i: (0, 0)),
            pl.BlockSpec((256, 10), lambda i: (0, 0)),
            pl.BlockSpec((1, 10), lambda i: (0, 0)),
        ],
        out_specs=pl.BlockSpec((_G, 10), lambda i: (0, 0)),
        out_shape=jax.ShapeDtypeStruct((_G, 10), jnp.float32),
        scratch_shapes=[
            pltpu.VMEM((_G, 256), jnp.float32),
            pltpu.VMEM((_G, 128), jnp.float32),
        ],
    )(h2way, acc2, batch.reshape(nblk, 1, BN), W21, b21.reshape(1, -1),
      W22, b22.reshape(1, -1), W_out, b_out.reshape(1, -1))


def kernel(x, edge_index, edge_attr, batch, W_e1, b_e1, W_e2, b_e2, W_c1e, b_c1e,
           W11, b11, W12, b12, W21, b21, W22, b22, W_out, b_out):
    src2 = edge_index[0].reshape(_E // 128, 128)
    dst2 = edge_index[1].reshape(_E // 128, 128)

    ea2, lin1 = _edge_encoder(edge_attr, W_e1, b_e1, W_e2, b_e2, W_c1e, b_c1e)
    acc1 = _scatter_phase(x, lin1, src2, dst2, feat_split=False)
    h2way = _node_mlp1(x, acc1, W11, b11, W12, b12)
    hflat = h2way.reshape(2 * _N, 128)
    acc2 = _scatter_phase(hflat, ea2, src2, dst2, feat_split=True)
    return _final_stage(h2way, acc2, batch, W21, b21, W22, b22, W_out, b_out)
